# baseline (speedup 1.0000x reference)
import functools

import jax
import jax.numpy as jnp
from jax import lax
from jax.experimental import pallas as pl
from jax.experimental.pallas import tpu as pltpu
from jax.experimental.pallas import tpu_sc as plsc

N_ROWS = 100000
N_FEAT = 8
N_TYPES = 128
N_WORKERS = 32
CHUNK = 400
N_CHUNKS = N_ROWS // CHUNK
N_SLOTS = -(-N_CHUNKS // N_WORKERS)
LAST_FULL_WID = N_CHUNKS - N_WORKERS * (N_SLOTS - 1)


def _make_kernel():
    mesh = plsc.VectorSubcoreMesh(core_axis_name="c", subcore_axis_name="s")

    @functools.partial(
        pl.kernel,
        mesh=mesh,
        compiler_params=pltpu.CompilerParams(needs_layout_passes=False),
        out_type=jax.ShapeDtypeStruct((N_ROWS, N_TYPES), jnp.float32),
        scratch_types=[
            pltpu.VMEM((CHUNK,), jnp.int32),
            pltpu.VMEM((CHUNK,), jnp.int32),
            pltpu.VMEM((CHUNK,), jnp.int32),
            pltpu.VMEM((CHUNK,), jnp.int32),
            pltpu.VMEM((CHUNK, N_TYPES), jnp.float32),
            pltpu.VMEM((CHUNK, N_TYPES), jnp.float32),
            pltpu.SemaphoreType.DMA,
            pltpu.SemaphoreType.DMA,
            pltpu.SemaphoreType.DMA,
            pltpu.SemaphoreType.DMA,
        ],
    )
    def onehot(idx_hbm, out_hbm, in0, in1, save0, save1,
               buf0, buf1, isem0, isem1, osem0, osem1):
        ins = (in0, in1)
        saves = (save0, save1)
        bufs = (buf0, buf1)
        isems = (isem0, isem1)
        osems = (osem0, osem1)

        wid = lax.axis_index("s") * 2 + lax.axis_index("c")
        lanes = lax.iota(jnp.int32, 16)
        ones_f = jnp.full((16,), 1.0, jnp.float32)
        zeros_f = jnp.zeros((16,), jnp.float32)

        in_descs = []
        out_descs = []
        for t in range(N_SLOTS):
            p = t % 2
            base = (wid + t * N_WORKERS) * CHUNK
            in_descs.append(pltpu.make_async_copy(
                idx_hbm.at[pl.ds(base, CHUNK)], ins[p], isems[p]))
            out_descs.append(pltpu.make_async_copy(
                bufs[p], out_hbm.at[pl.ds(base, CHUNK)], osems[p]))

        in_descs[0].start()
        in_descs[1].start()

        def _zero_row(r, _):
            for buf in bufs:
                for c in range(N_TYPES // 16):
                    buf[r, pl.ds(c * 16, 16)] = zeros_f
            return 0

        lax.fori_loop(0, CHUNK, _zero_row, 0)

        for t in range(N_SLOTS):
            p = t % 2
            in_v, save, buf = ins[p], saves[p], bufs[p]
            chunk = wid + t * N_WORKERS

            @pl.when(chunk < N_CHUNKS)
            def _(t=t, in_v=in_v, save=save, buf=buf):
                in_descs[t].wait()
                if t >= 2:
                    out_descs[t - 2].wait()
                    for g in range(CHUNK // 16):
                        rows = lanes + (g * 16)
                        cols = save[pl.ds(g * 16, 16)]
                        plsc.store_scatter(buf, [rows, cols], zeros_f)
                for g in range(CHUNK // 16):
                    rows = lanes + (g * 16)
                    cols = in_v[pl.ds(g * 16, 16)]
                    save[pl.ds(g * 16, 16)] = cols
                    plsc.store_scatter(buf, [rows, cols], ones_f)
                out_descs[t].start()

            if t + 2 < N_SLOTS:
                @pl.when(chunk + 2 * N_WORKERS < N_CHUNKS)
                def _(t=t):
                    in_descs[t + 2].start()

        @pl.when(wid >= LAST_FULL_WID)
        def _():
            out_descs[N_SLOTS - 3].wait()

        out_descs[N_SLOTS - 2].wait()

        @pl.when(wid < LAST_FULL_WID)
        def _():
            out_descs[N_SLOTS - 1].wait()

    return onehot


_onehot = _make_kernel()


@jax.jit
def kernel(node_feat):
    idx = node_feat[:, 0].astype(jnp.int32)
    return _onehot(idx)


# speedup vs baseline: 1.0298x; 1.0298x over previous
import functools

import jax
import jax.numpy as jnp
from jax import lax
from jax.experimental import pallas as pl
from jax.experimental.pallas import tpu as pltpu
from jax.experimental.pallas import tpu_sc as plsc

N_ROWS = 100000
N_FEAT = 8
N_TYPES = 128
N_WORKERS = 32
CHUNK = 160
N_CHUNKS = N_ROWS // CHUNK
N_SLOTS = -(-N_CHUNKS // N_WORKERS)
LAST_FULL_WID = N_CHUNKS - N_WORKERS * (N_SLOTS - 1)


def _make_kernel():
    mesh = plsc.VectorSubcoreMesh(core_axis_name="c", subcore_axis_name="s")

    @functools.partial(
        pl.kernel,
        mesh=mesh,
        compiler_params=pltpu.CompilerParams(needs_layout_passes=False),
        out_type=jax.ShapeDtypeStruct((N_ROWS, N_TYPES), jnp.float32),
        scratch_types=[
            pltpu.VMEM((CHUNK,), jnp.int32),
            pltpu.VMEM((CHUNK,), jnp.int32),
            pltpu.VMEM((CHUNK,), jnp.int32),
            pltpu.VMEM((CHUNK,), jnp.int32),
            pltpu.VMEM((CHUNK, N_TYPES), jnp.float32),
            pltpu.VMEM((CHUNK, N_TYPES), jnp.float32),
            pltpu.SemaphoreType.DMA,
            pltpu.SemaphoreType.DMA,
            pltpu.SemaphoreType.DMA,
            pltpu.SemaphoreType.DMA,
        ],
    )
    def onehot(idx_hbm, out_hbm, in0, in1, save0, save1,
               buf0, buf1, isem0, isem1, osem0, osem1):
        ins = (in0, in1)
        saves = (save0, save1)
        bufs = (buf0, buf1)
        isems = (isem0, isem1)
        osems = (osem0, osem1)

        wid = lax.axis_index("s") * 2 + lax.axis_index("c")
        lanes = lax.iota(jnp.int32, 16)
        ones_f = jnp.full((16,), 1.0, jnp.float32)
        zeros_f = jnp.zeros((16,), jnp.float32)

        in_descs = []
        out_descs = []
        for t in range(N_SLOTS):
            p = t % 2
            base = (wid + t * N_WORKERS) * CHUNK
            in_descs.append(pltpu.make_async_copy(
                idx_hbm.at[pl.ds(base, CHUNK)], ins[p], isems[p]))
            out_descs.append(pltpu.make_async_copy(
                bufs[p], out_hbm.at[pl.ds(base, CHUNK)], osems[p]))

        in_descs[0].start()
        in_descs[1].start()

        def _zero_row(r, _):
            for buf in bufs:
                for c in range(N_TYPES // 16):
                    buf[r, pl.ds(c * 16, 16)] = zeros_f
            return 0

        lax.fori_loop(0, CHUNK, _zero_row, 0)

        for t in range(N_SLOTS):
            p = t % 2
            in_v, save, buf = ins[p], saves[p], bufs[p]
            chunk = wid + t * N_WORKERS

            @pl.when(chunk < N_CHUNKS)
            def _(t=t, in_v=in_v, save=save, buf=buf):
                in_descs[t].wait()
                if t >= 2:
                    out_descs[t - 2].wait()
                    for g in range(CHUNK // 16):
                        rows = lanes + (g * 16)
                        cols = save[pl.ds(g * 16, 16)]
                        plsc.store_scatter(buf, [rows, cols], zeros_f)
                for g in range(CHUNK // 16):
                    rows = lanes + (g * 16)
                    cols = in_v[pl.ds(g * 16, 16)]
                    save[pl.ds(g * 16, 16)] = cols
                    plsc.store_scatter(buf, [rows, cols], ones_f)
                out_descs[t].start()

            if t + 2 < N_SLOTS:
                @pl.when(chunk + 2 * N_WORKERS < N_CHUNKS)
                def _(t=t):
                    in_descs[t + 2].start()

        @pl.when(wid >= LAST_FULL_WID)
        def _():
            out_descs[N_SLOTS - 3].wait()

        out_descs[N_SLOTS - 2].wait()

        @pl.when(wid < LAST_FULL_WID)
        def _():
            out_descs[N_SLOTS - 1].wait()

    return onehot


_onehot = _make_kernel()


@jax.jit
def kernel(node_feat):
    idx = node_feat[:, 0].astype(jnp.int32)
    return _onehot(idx)


# speedup vs baseline: 1.0438x; 1.0136x over previous
import functools

import jax
import jax.numpy as jnp
from jax import lax
from jax.experimental import pallas as pl
from jax.experimental.pallas import tpu as pltpu
from jax.experimental.pallas import tpu_sc as plsc

N_ROWS = 100000
N_FEAT = 8
N_TYPES = 128
N_WORKERS = 32
CHUNK = 160
N_CHUNKS = N_ROWS // CHUNK
N_SLOTS = -(-N_CHUNKS // N_WORKERS)
LAST_FULL_WID = N_CHUNKS - N_WORKERS * (N_SLOTS - 1)


def _make_kernel():
    mesh = plsc.VectorSubcoreMesh(core_axis_name="c", subcore_axis_name="s")

    @functools.partial(
        pl.kernel,
        mesh=mesh,
        compiler_params=pltpu.CompilerParams(needs_layout_passes=False),
        out_type=jax.ShapeDtypeStruct((N_ROWS, N_TYPES), jnp.float32),
        scratch_types=[
            pltpu.VMEM((CHUNK,), jnp.int32),
            pltpu.VMEM((CHUNK,), jnp.int32),
            pltpu.VMEM((CHUNK,), jnp.int32),
            pltpu.VMEM((CHUNK,), jnp.int32),
            pltpu.VMEM((CHUNK, N_TYPES), jnp.float32),
            pltpu.VMEM((CHUNK, N_TYPES), jnp.float32),
            pltpu.SemaphoreType.DMA,
            pltpu.SemaphoreType.DMA,
            pltpu.SemaphoreType.DMA,
            pltpu.SemaphoreType.DMA,
        ],
    )
    def onehot(idx_hbm, out_hbm, in0, in1, save0, save1,
               buf0, buf1, isem0, isem1, osem0, osem1):
        ins = (in0, in1)
        saves = (save0, save1)
        bufs = (buf0, buf1)
        isems = (isem0, isem1)
        osems = (osem0, osem1)

        wid = lax.axis_index("s") * 2 + lax.axis_index("c")
        lanes = lax.iota(jnp.int32, 16)
        ones_f = jnp.full((16,), 1.0, jnp.float32)
        zeros_f = jnp.zeros((16,), jnp.float32)

        in_descs = []
        out_descs = []
        for t in range(N_SLOTS):
            p = t % 2
            base = (wid + t * N_WORKERS) * CHUNK
            in_descs.append(pltpu.make_async_copy(
                idx_hbm.at[pl.ds(base, CHUNK)], ins[p], isems[p]))
            out_descs.append(pltpu.make_async_copy(
                bufs[p], out_hbm.at[pl.ds(base, CHUNK)], osems[p]))

        in_descs[0].start()
        in_descs[1].start()

        def _zero_row(r, _):
            for buf in bufs:
                for c in range(N_TYPES // 16):
                    buf[r, pl.ds(c * 16, 16)] = zeros_f
            return 0

        lax.fori_loop(0, CHUNK, _zero_row, 0)

        for t in range(N_SLOTS):
            p = t % 2
            in_v, save, buf = ins[p], saves[p], bufs[p]
            chunk = wid + t * N_WORKERS

            @pl.when(chunk < N_CHUNKS)
            def _(t=t, in_v=in_v, save=save, buf=buf):
                in_descs[t].wait()
                cols_vecs = [in_v[pl.ds(g * 16, 16)]
                             for g in range(CHUNK // 16)]
                if t >= 2:
                    out_descs[t - 2].wait()
                    for g in range(CHUNK // 16):
                        rows = lanes + (g * 16)
                        old_cols = save[pl.ds(g * 16, 16)]
                        plsc.store_scatter(buf, [rows, old_cols], zeros_f)
                for g in range(CHUNK // 16):
                    rows = lanes + (g * 16)
                    save[pl.ds(g * 16, 16)] = cols_vecs[g]
                    plsc.store_scatter(buf, [rows, cols_vecs[g]], ones_f)
                pltpu.touch(buf)
                out_descs[t].start()

            if t + 2 < N_SLOTS:
                @pl.when(chunk + 2 * N_WORKERS < N_CHUNKS)
                def _(t=t, in_v=in_v):
                    pltpu.touch(in_v)
                    in_descs[t + 2].start()

        @pl.when(wid >= LAST_FULL_WID)
        def _():
            out_descs[N_SLOTS - 3].wait()

        out_descs[N_SLOTS - 2].wait()

        @pl.when(wid < LAST_FULL_WID)
        def _():
            out_descs[N_SLOTS - 1].wait()

    return onehot


_onehot = _make_kernel()


@jax.jit
def kernel(node_feat):
    idx = node_feat[:, 0].astype(jnp.int32)
    return _onehot(idx)


# speedup vs baseline: 1.0631x; 1.0184x over previous
import functools

import jax
import jax.numpy as jnp
from jax import lax
from jax.experimental import pallas as pl
from jax.experimental.pallas import tpu as pltpu
from jax.experimental.pallas import tpu_sc as plsc

N_ROWS = 100000
N_FEAT = 8
N_TYPES = 128
N_WORKERS = 32
CHUNK = 160
N_CHUNKS = N_ROWS // CHUNK
N_SLOTS = -(-N_CHUNKS // N_WORKERS)
LAST_FULL_WID = N_CHUNKS - N_WORKERS * (N_SLOTS - 1)


def _make_kernel():
    mesh = plsc.VectorSubcoreMesh(core_axis_name="c", subcore_axis_name="s")

    @functools.partial(
        pl.kernel,
        mesh=mesh,
        compiler_params=pltpu.CompilerParams(needs_layout_passes=False),
        out_type=jax.ShapeDtypeStruct((N_ROWS, N_TYPES), jnp.float32),
        scratch_types=(
            [pltpu.VMEM((CHUNK,), jnp.int32) for _ in range(N_SLOTS)]
            + [pltpu.VMEM((CHUNK, N_TYPES), jnp.float32)] * 2
            + [pltpu.SemaphoreType.DMA] * (N_SLOTS + 2)
        ),
    )
    def onehot(idx_hbm, out_hbm, *scratch):
        ins = scratch[:N_SLOTS]
        bufs = scratch[N_SLOTS:N_SLOTS + 2]
        isems = scratch[N_SLOTS + 2:2 * N_SLOTS + 2]
        osems = scratch[2 * N_SLOTS + 2:]

        wid = lax.axis_index("s") * 2 + lax.axis_index("c")
        lanes = lax.iota(jnp.int32, 16)
        ones_f = jnp.full((16,), 1.0, jnp.float32)
        zeros_f = jnp.zeros((16,), jnp.float32)

        in_descs = []
        out_descs = []
        for t in range(N_SLOTS):
            base = (wid + t * N_WORKERS) * CHUNK
            in_descs.append(pltpu.make_async_copy(
                idx_hbm.at[pl.ds(base, CHUNK)], ins[t], isems[t]))
            out_descs.append(pltpu.make_async_copy(
                bufs[t % 2], out_hbm.at[pl.ds(base, CHUNK)], osems[t % 2]))

        for t in range(N_SLOTS - 1):
            in_descs[t].start()

        @pl.when(wid < LAST_FULL_WID)
        def _():
            in_descs[N_SLOTS - 1].start()

        def _zero_row(r, _):
            for buf in bufs:
                for c in range(N_TYPES // 16):
                    buf[r, pl.ds(c * 16, 16)] = zeros_f
            return 0

        lax.fori_loop(0, CHUNK, _zero_row, 0)

        for t in range(N_SLOTS):
            buf = bufs[t % 2]
            chunk = wid + t * N_WORKERS

            @pl.when(chunk < N_CHUNKS)
            def _(t=t, buf=buf):
                in_descs[t].wait()
                if t >= 2:
                    out_descs[t - 2].wait()
                    for g in range(CHUNK // 16):
                        rows = lanes + (g * 16)
                        old_cols = ins[t - 2][pl.ds(g * 16, 16)]
                        plsc.store_scatter(buf, [rows, old_cols], zeros_f)
                for g in range(CHUNK // 16):
                    rows = lanes + (g * 16)
                    cols = ins[t][pl.ds(g * 16, 16)]
                    plsc.store_scatter(buf, [rows, cols], ones_f)
                pltpu.touch(buf)
                out_descs[t].start()

        @pl.when(wid >= LAST_FULL_WID)
        def _():
            out_descs[N_SLOTS - 3].wait()

        out_descs[N_SLOTS - 2].wait()

        @pl.when(wid < LAST_FULL_WID)
        def _():
            out_descs[N_SLOTS - 1].wait()

    return onehot


_onehot = _make_kernel()


@jax.jit
def kernel(node_feat):
    idx = node_feat[:, 0].astype(jnp.int32)
    return _onehot(idx)
